# baseline (speedup 1.0000x reference)
import jax
import jax.numpy as jnp
from jax.experimental import pallas as pl
from jax.experimental.pallas import tpu as pltpu


_BC = 512


def _body(x_ref, t_ref, out_ref):
    i = pl.program_id(0)
    nb = pl.num_programs(0)
    x = x_ref[...]
    t = t_ref[...]
    masked = jnp.where(t > 0.0, x, -jnp.inf)
    m = jnp.max(masked, axis=0, keepdims=True)
    hp = jnp.max(t, axis=0, keepdims=True) > 0.0
    sig = jnp.clip(jax.nn.sigmoid(m), 1e-6, 1.0 - 1e-6)
    li = jnp.where(hp, -jnp.log(sig), 0.0)
    part = jnp.sum(li, axis=(0, 1), keepdims=True)

    @pl.when(i == 0)
    def _():
        out_ref[...] = jnp.zeros_like(out_ref)

    out_ref[...] += part

    @pl.when(i == nb - 1)
    def _():
        out_ref[...] = out_ref[...] * (1.0 / (nb * _BC))


@jax.jit
def kernel(input, target):
    B, C = input.shape
    xT = input.T
    tT = target.T
    nb = B // _BC
    out = pl.pallas_call(
        _body,
        grid=(nb,),
        in_specs=[
            pl.BlockSpec((C, _BC), lambda i: (0, i)),
            pl.BlockSpec((C, _BC), lambda i: (0, i)),
        ],
        out_specs=pl.BlockSpec((1, 1), lambda i: (0, 0)),
        out_shape=jax.ShapeDtypeStruct((1, 1), jnp.float32),
    )(xT, tT)
    return out[0, 0]


# speedup vs baseline: 1.1647x; 1.1647x over previous
import jax
import jax.numpy as jnp
from jax.experimental import pallas as pl
from jax.experimental.pallas import tpu as pltpu


_BC = 1024
_WAYS = 2


def _chunk(x, t):
    masked = jnp.where(t > 0.0, x, -jnp.inf)
    m = jnp.max(masked, axis=0, keepdims=True)
    hp = jnp.max(t, axis=0, keepdims=True) > 0.0
    sig = jnp.clip(jax.nn.sigmoid(m), 1e-6, 1.0 - 1e-6)
    li = jnp.where(hp, -jnp.log(sig), 0.0)
    return jnp.sum(li, axis=(0, 1), keepdims=True)


def _body(x0_ref, x1_ref, t0_ref, t1_ref, out_ref):
    i = pl.program_id(0)
    nb = pl.num_programs(0)
    part = _chunk(x0_ref[...], t0_ref[...]) + _chunk(x1_ref[...], t1_ref[...])

    @pl.when(i == 0)
    def _():
        out_ref[...] = jnp.zeros_like(out_ref)

    out_ref[...] += part

    @pl.when(i == nb - 1)
    def _():
        out_ref[...] = out_ref[...] * (1.0 / (nb * _BC * _WAYS))


@jax.jit
def kernel(input, target):
    B, C = input.shape
    xT = input.T
    tT = target.T
    nb = B // (_BC * _WAYS)

    def mk(w):
        return pl.BlockSpec((C, _BC), lambda i, w=w: (0, w * nb + i))

    specs = [mk(w) for w in range(_WAYS)]
    out = pl.pallas_call(
        _body,
        grid=(nb,),
        in_specs=specs + specs,
        out_specs=pl.BlockSpec((1, 1), lambda i: (0, 0)),
        out_shape=jax.ShapeDtypeStruct((1, 1), jnp.float32),
    )(xT, xT, tT, tT)
    return out[0, 0]
